# baseline (speedup 1.0000x reference)
import functools

import jax
import jax.numpy as jnp
from jax import lax
from jax.experimental import pallas as pl
from jax.experimental.pallas import tpu as pltpu
from jax.experimental.pallas import tpu_sc as plsc

N = 10000
NPAD = 10240
E = 320000
NCORES = 2
NSUB = 16
NTILES = NCORES * NSUB
EPT = E // NTILES
CHUNK = 125
NCHUNK = EPT // CHUNK
STAGE = 40
NSTAGE = NCHUNK // STAGE
RPT = NPAD // NSUB
ZROWS = 128
ZCH = RPT // ZROWS

BM = 2000


def _fill2d(ref, nrows, ncols, value):
    per_row = ncols // 16
    vec = jnp.full((16,), value, dtype=jnp.float32)

    def body(i, c):
        r = i // per_row
        k = (i % per_row) * 16
        ref[r, pl.ds(k, 16)] = vec
        return c

    lax.fori_loop(0, nrows * per_row, body, 0)


def _fill1d(ref, n, value):
    vec = jnp.full((16,), value, dtype=jnp.float32)

    def body(i, c):
        ref[pl.ds(i * 16, 16)] = vec
        return c

    lax.fori_loop(0, n // 16, body, 0)


def _make_sc_layer(D, with_deg):
    def body(*refs):
        if with_deg:
            (y_hbm, row_hbm, col_hbm, part_hbm, dega_hbm, degb_hbm,
             agg_sh, deg_sh, rows0_v, rows1_v, ridx_v, cidx_v, dx0_v, dx1_v,
             ones_v, zdeg_v, gsem0, gsem1, dsem0, dsem1, hsem0,
             hsem1, isem0, isem1, jsem0, jsem1) = refs
        else:
            (y_hbm, row_hbm, col_hbm, part_hbm,
             agg_sh, rows0_v, rows1_v, ridx_v, cidx_v, gsem0, gsem1,
             hsem0, hsem1, isem0, isem1, jsem0, jsem1) = refs
        rows_v = rows0_v

        cid = lax.axis_index("c")
        sid = lax.axis_index("s")
        wid = cid * NSUB + sid

        _fill2d(rows_v, ZROWS, D, 0.0)
        for k in range(ZCH):
            pltpu.async_copy(
                rows_v, agg_sh.at[pl.ds(sid * RPT + k * ZROWS, ZROWS)], gsem0)
        if with_deg:
            _fill1d(ones_v, ZROWS, 1.0)
            _fill1d(zdeg_v, RPT, 0.0)
            pltpu.async_copy(zdeg_v, deg_sh.at[pl.ds(sid * RPT, RPT)], gsem1)
        for k in range(ZCH):
            pltpu.make_async_copy(
                rows_v, agg_sh.at[pl.ds(sid * RPT + k * ZROWS, ZROWS)],
                gsem0).wait()
        if with_deg:
            pltpu.make_async_copy(
                zdeg_v, deg_sh.at[pl.ds(sid * RPT, RPT)], gsem1).wait()
        plsc.subcore_barrier()

        rows = (rows0_v.at[pl.ds(0, CHUNK)], rows1_v.at[pl.ds(0, CHUNK)])
        splits = ((0, 32), (32, 32), (64, 32), (96, 29))
        rowsS = tuple(tuple(rv.at[pl.ds(o, n)] for (o, n) in splits)
                      for rv in (rows0_v, rows1_v))
        gsemsS = ((gsem0, hsem0, isem0, jsem0), (gsem1, hsem1, isem1, jsem1))
        if with_deg:
            dxs = (dx0_v, dx1_v)
            dsems = (dsem0, dsem1)
            ones = ones_v.at[pl.ds(0, CHUNK)]

        for q in range(NSTAGE):
            cbase = wid * NCHUNK + q * STAGE
            pltpu.sync_copy(row_hbm.at[pl.ds(cbase, STAGE)], ridx_v)
            pltpu.sync_copy(col_hbm.at[pl.ds(cbase, STAGE)], cidx_v)
            for (o, n), rs, sm in zip(splits, rowsS[0], gsemsS[0]):
                pltpu.async_copy(y_hbm.at[ridx_v.at[0, pl.ds(o, n)]], rs, sm)
            if with_deg:
                pltpu.async_copy(col_hbm.at[cbase], dx0_v, dsem0)

            def do_chunk(j, b, prefetch):
                for (o, n), rs, sm in zip(splits, rowsS[b], gsemsS[b]):
                    pltpu.make_async_copy(y_hbm.at[ridx_v.at[j, pl.ds(o, n)]],
                                          rs, sm).wait()
                if prefetch:
                    for (o, n), rs, sm in zip(splits, rowsS[1 - b],
                                              gsemsS[1 - b]):
                        pltpu.async_copy(y_hbm.at[ridx_v.at[j + 1,
                                                            pl.ds(o, n)]],
                                         rs, sm)
                    if with_deg:
                        pltpu.async_copy(col_hbm.at[cbase + j + 1],
                                         dxs[1 - b], dsems[1 - b])
                pltpu.sync_copy(rows[b], agg_sh.at[cidx_v.at[j]], add=True)
                if with_deg:
                    pltpu.make_async_copy(col_hbm.at[cbase + j], dxs[b],
                                          dsems[b]).wait()
                    pltpu.sync_copy(ones, deg_sh.at[dxs[b]], add=True)

            def pair(i, c):
                do_chunk(2 * i, 0, True)
                do_chunk(2 * i + 1, 1, True)
                return c

            lax.fori_loop(0, STAGE // 2 - 1, pair, 0)
            do_chunk(STAGE - 2, 0, True)
            do_chunk(STAGE - 1, 1, False)
        plsc.subcore_barrier()

        pltpu.sync_copy(agg_sh.at[pl.ds(sid * RPT, RPT)],
                        part_hbm.at[cid, pl.ds(sid * RPT, RPT)])
        if with_deg:
            @pl.when(cid == 0)
            def _():
                pltpu.sync_copy(deg_sh.at[pl.ds(sid * RPT, RPT)],
                                dega_hbm.at[pl.ds(sid * RPT, RPT)])

            @pl.when(cid == 1)
            def _():
                pltpu.sync_copy(deg_sh.at[pl.ds(sid * RPT, RPT)],
                                degb_hbm.at[pl.ds(sid * RPT, RPT)])

    out_type = [jax.ShapeDtypeStruct((NCORES, NPAD, D), jnp.float32)]
    scratch = [
        pltpu.VMEM_SHARED((NPAD, D), jnp.float32),
    ]
    if with_deg:
        out_type.append(jax.ShapeDtypeStruct((NPAD,), jnp.float32))
        out_type.append(jax.ShapeDtypeStruct((NPAD,), jnp.float32))
        scratch.append(pltpu.VMEM_SHARED((NPAD,), jnp.float32))
    scratch += [
        pltpu.VMEM((ZROWS, D), jnp.float32),
        pltpu.VMEM((ZROWS, D), jnp.float32),
        pltpu.VMEM((STAGE, CHUNK), jnp.int32),
        pltpu.VMEM((STAGE, CHUNK), jnp.int32),
    ]
    if with_deg:
        scratch += [
            pltpu.VMEM((CHUNK,), jnp.int32),
            pltpu.VMEM((CHUNK,), jnp.int32),
            pltpu.VMEM((ZROWS,), jnp.float32),
            pltpu.VMEM((RPT,), jnp.float32),
        ]
    scratch.append(pltpu.SemaphoreType.DMA)
    scratch.append(pltpu.SemaphoreType.DMA)
    if with_deg:
        scratch.append(pltpu.SemaphoreType.DMA)
        scratch.append(pltpu.SemaphoreType.DMA)
    for _ in range(6):
        scratch.append(pltpu.SemaphoreType.DMA)

    mesh = plsc.VectorSubcoreMesh(core_axis_name="c", subcore_axis_name="s",
                                  num_cores=NCORES, num_subcores=NSUB)
    return pl.kernel(body, out_type=out_type, mesh=mesh, scratch_types=scratch)


@functools.lru_cache(maxsize=None)
def _sc_layer(D, with_deg):
    return _make_sc_layer(D, with_deg)


def _tc_first(x, w_out, w_root, b):
    def body(x_ref, wo_ref, wr_ref, b_ref, y_ref, r_ref):
        xb = x_ref[...]
        y_ref[...] = jnp.dot(xb, wo_ref[...], preferred_element_type=jnp.float32)
        r_ref[...] = (jnp.dot(xb, wr_ref[...], preferred_element_type=jnp.float32)
                      + b_ref[...])

    D = w_out.shape[1]
    return pl.pallas_call(
        body,
        grid=(N // BM,),
        in_specs=[
            pl.BlockSpec((BM, 128), lambda i: (i, 0)),
            pl.BlockSpec((128, D), lambda i: (0, 0)),
            pl.BlockSpec((128, D), lambda i: (0, 0)),
            pl.BlockSpec((1, D), lambda i: (0, 0)),
        ],
        out_specs=[pl.BlockSpec((BM, D), lambda i: (i, 0)),
                   pl.BlockSpec((BM, D), lambda i: (i, 0))],
        out_shape=[jax.ShapeDtypeStruct((N, D), jnp.float32)] * 2,
    )(x, w_out, w_root, b.reshape(1, D))


def _tc_mid1(p, dega, degb, r_prev, w_out, w_root, b):
    def body(p_ref, da_ref, db_ref, rp_ref, wo_ref, wr_ref, b_ref,
             y_ref, r_ref, dinv_ref):
        deg = da_ref[...] + db_ref[...]
        dinv = 1.0 / jnp.maximum(deg, 1.0)
        dinv_ref[...] = jnp.broadcast_to(dinv, dinv_ref.shape)
        h = jnp.maximum((p_ref[0] + p_ref[1]) * dinv + rp_ref[...], 0.0)
        y_ref[...] = jnp.dot(h, wo_ref[...], preferred_element_type=jnp.float32)
        r_ref[...] = (jnp.dot(h, wr_ref[...], preferred_element_type=jnp.float32)
                      + b_ref[...])

    D = w_out.shape[1]
    return pl.pallas_call(
        body,
        grid=(N // BM,),
        in_specs=[
            pl.BlockSpec((2, BM, 128), lambda i: (0, i, 0)),
            pl.BlockSpec((BM, 1), lambda i: (i, 0)),
            pl.BlockSpec((BM, 1), lambda i: (i, 0)),
            pl.BlockSpec((BM, 128), lambda i: (i, 0)),
            pl.BlockSpec((128, D), lambda i: (0, 0)),
            pl.BlockSpec((128, D), lambda i: (0, 0)),
            pl.BlockSpec((1, D), lambda i: (0, 0)),
        ],
        out_specs=[pl.BlockSpec((BM, D), lambda i: (i, 0)),
                   pl.BlockSpec((BM, D), lambda i: (i, 0)),
                   pl.BlockSpec((BM, 16), lambda i: (i, 0))],
        out_shape=[jax.ShapeDtypeStruct((N, D), jnp.float32),
                   jax.ShapeDtypeStruct((N, D), jnp.float32),
                   jax.ShapeDtypeStruct((N, 16), jnp.float32)],
    )(p, dega, degb, r_prev, w_out, w_root, b.reshape(1, D))


def _tc_mid2(p, dinv16, r_prev, w_out, w_root, b):
    def body(p_ref, dv_ref, rp_ref, wo_ref, wr_ref, b_ref, y_ref, r_ref):
        dinv = dv_ref[:, 0:1]
        h = jnp.maximum((p_ref[0] + p_ref[1]) * dinv + rp_ref[...], 0.0)
        y_ref[...] = jnp.dot(h, wo_ref[...], preferred_element_type=jnp.float32)
        r_ref[...] = (jnp.dot(h, wr_ref[...], preferred_element_type=jnp.float32)
                      + b_ref[...])

    D = w_out.shape[1]
    return pl.pallas_call(
        body,
        grid=(N // BM,),
        in_specs=[
            pl.BlockSpec((2, BM, 128), lambda i: (0, i, 0)),
            pl.BlockSpec((BM, 16), lambda i: (i, 0)),
            pl.BlockSpec((BM, 128), lambda i: (i, 0)),
            pl.BlockSpec((128, D), lambda i: (0, 0)),
            pl.BlockSpec((128, D), lambda i: (0, 0)),
            pl.BlockSpec((1, D), lambda i: (0, 0)),
        ],
        out_specs=[pl.BlockSpec((BM, D), lambda i: (i, 0)),
                   pl.BlockSpec((BM, D), lambda i: (i, 0))],
        out_shape=[jax.ShapeDtypeStruct((N, D), jnp.float32)] * 2,
    )(p, dinv16, r_prev, w_out, w_root, b.reshape(1, D))


def _tc_final(p, dinv16, r_prev, d_out):
    def body(p_ref, dv_ref, rp_ref, o_ref):
        h = (p_ref[0] + p_ref[1]) * dv_ref[:, 0:1] + rp_ref[...]
        col = lax.broadcasted_iota(jnp.int32, h.shape, 1)
        hm = jnp.where(col < d_out, h, jnp.float32(-1e30))
        m = jnp.max(hm, axis=-1, keepdims=True)
        lse = jnp.log(jnp.sum(jnp.exp(hm - m), axis=-1, keepdims=True))
        o_ref[...] = (h - m - lse)[:, :d_out]

    D = p.shape[2]
    return pl.pallas_call(
        body,
        grid=(N // BM,),
        in_specs=[
            pl.BlockSpec((2, BM, D), lambda i: (0, i, 0)),
            pl.BlockSpec((BM, 16), lambda i: (i, 0)),
            pl.BlockSpec((BM, D), lambda i: (i, 0)),
        ],
        out_specs=pl.BlockSpec((BM, d_out), lambda i: (i, 0)),
        out_shape=jax.ShapeDtypeStruct((N, d_out), jnp.float32),
    )(p, dinv16, r_prev)


def kernel(x, edge_index, W_out_0, b_out_0, W_root_0, W_out_1, b_out_1,
           W_root_1, W_out_2, b_out_2, W_root_2):
    row2 = edge_index[0].reshape(E // CHUNK, CHUNK)
    col2 = edge_index[1].reshape(E // CHUNK, CHUNK)

    y0, r0 = _tc_first(x, W_out_0, W_root_0, b_out_0)
    p0, dega, degb = _sc_layer(128, True)(y0, row2, col2)
    y1, r1, dinv16 = _tc_mid1(p0, dega.reshape(NPAD, 1), degb.reshape(NPAD, 1),
                              r0, W_out_1, W_root_1, b_out_1)
    (p1,) = _sc_layer(128, False)(y1, row2, col2)
    w2o = jnp.pad(W_out_2, ((0, 0), (0, 64)))
    w2r = jnp.pad(W_root_2, ((0, 0), (0, 64)))
    b2 = jnp.pad(b_out_2, (0, 64))
    y2, r2 = _tc_mid2(p1, dinv16, r1, w2o, w2r, b2)
    (p2,) = _sc_layer(128, False)(y2, row2, col2)
    return _tc_final(p2, dinv16, r2, 64)


# speedup vs baseline: 1.0145x; 1.0145x over previous
import functools

import jax
import jax.numpy as jnp
from jax import lax
from jax.experimental import pallas as pl
from jax.experimental.pallas import tpu as pltpu
from jax.experimental.pallas import tpu_sc as plsc

N = 10000
NPAD = 10240
E = 320000
NCORES = 2
NSUB = 16
NTILES = NCORES * NSUB
EPT = E // NTILES
CHUNK = 125
NCHUNK = EPT // CHUNK
STAGE = 40
NSTAGE = NCHUNK // STAGE
RPT = NPAD // NSUB
ZROWS = 128
ZCH = RPT // ZROWS

BM = 2000


def _fill2d(ref, nrows, ncols, value):
    per_row = ncols // 16
    vec = jnp.full((16,), value, dtype=jnp.float32)

    def body(i, c):
        r = i // per_row
        k = (i % per_row) * 16
        ref[r, pl.ds(k, 16)] = vec
        return c

    lax.fori_loop(0, nrows * per_row, body, 0)


def _fill1d(ref, n, value):
    vec = jnp.full((16,), value, dtype=jnp.float32)

    def body(i, c):
        ref[pl.ds(i * 16, 16)] = vec
        return c

    lax.fori_loop(0, n // 16, body, 0)


def _make_sc_layer(D, with_deg):
    def body(*refs):
        if with_deg:
            (y_hbm, row_hbm, col_hbm, part_hbm, dega_hbm, degb_hbm,
             agg_sh, deg_sh, rows0_v, rows1_v, ridx_v, cidx_v, dx0_v, dx1_v,
             ones_v, zdeg_v, gsem0, gsem1, dsem0, dsem1, hsem0,
             hsem1) = refs
        else:
            (y_hbm, row_hbm, col_hbm, part_hbm,
             agg_sh, rows0_v, rows1_v, ridx_v, cidx_v, gsem0, gsem1,
             hsem0, hsem1) = refs
        rows_v = rows0_v

        cid = lax.axis_index("c")
        sid = lax.axis_index("s")
        wid = cid * NSUB + sid

        _fill2d(rows_v, ZROWS, D, 0.0)
        for k in range(ZCH):
            pltpu.async_copy(
                rows_v, agg_sh.at[pl.ds(sid * RPT + k * ZROWS, ZROWS)], gsem0)
        if with_deg:
            _fill1d(ones_v, ZROWS, 1.0)
            _fill1d(zdeg_v, RPT, 0.0)
            pltpu.async_copy(zdeg_v, deg_sh.at[pl.ds(sid * RPT, RPT)], gsem1)
        for k in range(ZCH):
            pltpu.make_async_copy(
                rows_v, agg_sh.at[pl.ds(sid * RPT + k * ZROWS, ZROWS)],
                gsem0).wait()
        if with_deg:
            pltpu.make_async_copy(
                zdeg_v, deg_sh.at[pl.ds(sid * RPT, RPT)], gsem1).wait()
        plsc.subcore_barrier()

        rows = (rows0_v.at[pl.ds(0, CHUNK)], rows1_v.at[pl.ds(0, CHUNK)])
        rowsA = (rows0_v.at[pl.ds(0, 64)], rows1_v.at[pl.ds(0, 64)])
        rowsB = (rows0_v.at[pl.ds(64, 61)], rows1_v.at[pl.ds(64, 61)])
        gsems = (gsem0, gsem1)
        hsems = (hsem0, hsem1)
        if with_deg:
            dxs = (dx0_v, dx1_v)
            dsems = (dsem0, dsem1)
            ones = ones_v.at[pl.ds(0, CHUNK)]

        for q in range(NSTAGE):
            cbase = wid * NCHUNK + q * STAGE
            pltpu.sync_copy(row_hbm.at[pl.ds(cbase, STAGE)], ridx_v)
            pltpu.sync_copy(col_hbm.at[pl.ds(cbase, STAGE)], cidx_v)
            pltpu.async_copy(y_hbm.at[ridx_v.at[0, pl.ds(0, 64)]],
                             rowsA[0], gsem0)
            pltpu.async_copy(y_hbm.at[ridx_v.at[0, pl.ds(64, 61)]],
                             rowsB[0], hsem0)
            if with_deg:
                pltpu.async_copy(col_hbm.at[cbase], dx0_v, dsem0)

            def do_chunk(j, b, prefetch):
                pltpu.make_async_copy(y_hbm.at[ridx_v.at[j, pl.ds(0, 64)]],
                                      rowsA[b], gsems[b]).wait()
                pltpu.make_async_copy(y_hbm.at[ridx_v.at[j, pl.ds(64, 61)]],
                                      rowsB[b], hsems[b]).wait()
                if prefetch:
                    pltpu.async_copy(y_hbm.at[ridx_v.at[j + 1, pl.ds(0, 64)]],
                                     rowsA[1 - b], gsems[1 - b])
                    pltpu.async_copy(y_hbm.at[ridx_v.at[j + 1, pl.ds(64, 61)]],
                                     rowsB[1 - b], hsems[1 - b])
                    if with_deg:
                        pltpu.async_copy(col_hbm.at[cbase + j + 1],
                                         dxs[1 - b], dsems[1 - b])
                pltpu.sync_copy(rows[b], agg_sh.at[cidx_v.at[j]], add=True)
                if with_deg:
                    pltpu.make_async_copy(col_hbm.at[cbase + j], dxs[b],
                                          dsems[b]).wait()
                    pltpu.sync_copy(ones, deg_sh.at[dxs[b]], add=True)

            def pair(i, c):
                do_chunk(2 * i, 0, True)
                do_chunk(2 * i + 1, 1, True)
                return c

            lax.fori_loop(0, STAGE // 2 - 1, pair, 0)
            do_chunk(STAGE - 2, 0, True)
            do_chunk(STAGE - 1, 1, False)
        plsc.subcore_barrier()

        pltpu.sync_copy(agg_sh.at[pl.ds(sid * RPT, RPT)],
                        part_hbm.at[cid, pl.ds(sid * RPT, RPT)])
        if with_deg:
            @pl.when(cid == 0)
            def _():
                pltpu.sync_copy(deg_sh.at[pl.ds(sid * RPT, RPT)],
                                dega_hbm.at[pl.ds(sid * RPT, RPT)])

            @pl.when(cid == 1)
            def _():
                pltpu.sync_copy(deg_sh.at[pl.ds(sid * RPT, RPT)],
                                degb_hbm.at[pl.ds(sid * RPT, RPT)])

    out_type = [jax.ShapeDtypeStruct((NCORES, NPAD, D), jnp.float32)]
    scratch = [
        pltpu.VMEM_SHARED((NPAD, D), jnp.float32),
    ]
    if with_deg:
        out_type.append(jax.ShapeDtypeStruct((NPAD,), jnp.float32))
        out_type.append(jax.ShapeDtypeStruct((NPAD,), jnp.float32))
        scratch.append(pltpu.VMEM_SHARED((NPAD,), jnp.float32))
    scratch += [
        pltpu.VMEM((ZROWS, D), jnp.float32),
        pltpu.VMEM((ZROWS, D), jnp.float32),
        pltpu.VMEM((STAGE, CHUNK), jnp.int32),
        pltpu.VMEM((STAGE, CHUNK), jnp.int32),
    ]
    if with_deg:
        scratch += [
            pltpu.VMEM((CHUNK,), jnp.int32),
            pltpu.VMEM((CHUNK,), jnp.int32),
            pltpu.VMEM((ZROWS,), jnp.float32),
            pltpu.VMEM((RPT,), jnp.float32),
        ]
    scratch.append(pltpu.SemaphoreType.DMA)
    scratch.append(pltpu.SemaphoreType.DMA)
    if with_deg:
        scratch.append(pltpu.SemaphoreType.DMA)
        scratch.append(pltpu.SemaphoreType.DMA)
    scratch.append(pltpu.SemaphoreType.DMA)
    scratch.append(pltpu.SemaphoreType.DMA)

    mesh = plsc.VectorSubcoreMesh(core_axis_name="c", subcore_axis_name="s",
                                  num_cores=NCORES, num_subcores=NSUB)
    return pl.kernel(body, out_type=out_type, mesh=mesh, scratch_types=scratch)


@functools.lru_cache(maxsize=None)
def _sc_layer(D, with_deg):
    return _make_sc_layer(D, with_deg)


def _tc_first(x, w_out, w_root, b):
    def body(x_ref, wo_ref, wr_ref, b_ref, y_ref, r_ref):
        xb = x_ref[...]
        y_ref[...] = jnp.dot(xb, wo_ref[...], preferred_element_type=jnp.float32)
        r_ref[...] = (jnp.dot(xb, wr_ref[...], preferred_element_type=jnp.float32)
                      + b_ref[...])

    D = w_out.shape[1]
    return pl.pallas_call(
        body,
        grid=(N // BM,),
        in_specs=[
            pl.BlockSpec((BM, 128), lambda i: (i, 0)),
            pl.BlockSpec((128, D), lambda i: (0, 0)),
            pl.BlockSpec((128, D), lambda i: (0, 0)),
            pl.BlockSpec((1, D), lambda i: (0, 0)),
        ],
        out_specs=[pl.BlockSpec((BM, D), lambda i: (i, 0)),
                   pl.BlockSpec((BM, D), lambda i: (i, 0))],
        out_shape=[jax.ShapeDtypeStruct((N, D), jnp.float32)] * 2,
    )(x, w_out, w_root, b.reshape(1, D))


def _tc_mid1(p, dega, degb, r_prev, w_out, w_root, b):
    def body(p_ref, da_ref, db_ref, rp_ref, wo_ref, wr_ref, b_ref,
             y_ref, r_ref, dinv_ref):
        deg = da_ref[...] + db_ref[...]
        dinv = 1.0 / jnp.maximum(deg, 1.0)
        dinv_ref[...] = jnp.broadcast_to(dinv, dinv_ref.shape)
        h = jnp.maximum((p_ref[0] + p_ref[1]) * dinv + rp_ref[...], 0.0)
        y_ref[...] = jnp.dot(h, wo_ref[...], preferred_element_type=jnp.float32)
        r_ref[...] = (jnp.dot(h, wr_ref[...], preferred_element_type=jnp.float32)
                      + b_ref[...])

    D = w_out.shape[1]
    return pl.pallas_call(
        body,
        grid=(N // BM,),
        in_specs=[
            pl.BlockSpec((2, BM, 128), lambda i: (0, i, 0)),
            pl.BlockSpec((BM, 1), lambda i: (i, 0)),
            pl.BlockSpec((BM, 1), lambda i: (i, 0)),
            pl.BlockSpec((BM, 128), lambda i: (i, 0)),
            pl.BlockSpec((128, D), lambda i: (0, 0)),
            pl.BlockSpec((128, D), lambda i: (0, 0)),
            pl.BlockSpec((1, D), lambda i: (0, 0)),
        ],
        out_specs=[pl.BlockSpec((BM, D), lambda i: (i, 0)),
                   pl.BlockSpec((BM, D), lambda i: (i, 0)),
                   pl.BlockSpec((BM, 16), lambda i: (i, 0))],
        out_shape=[jax.ShapeDtypeStruct((N, D), jnp.float32),
                   jax.ShapeDtypeStruct((N, D), jnp.float32),
                   jax.ShapeDtypeStruct((N, 16), jnp.float32)],
    )(p, dega, degb, r_prev, w_out, w_root, b.reshape(1, D))


def _tc_mid2(p, dinv16, r_prev, w_out, w_root, b):
    def body(p_ref, dv_ref, rp_ref, wo_ref, wr_ref, b_ref, y_ref, r_ref):
        dinv = dv_ref[:, 0:1]
        h = jnp.maximum((p_ref[0] + p_ref[1]) * dinv + rp_ref[...], 0.0)
        y_ref[...] = jnp.dot(h, wo_ref[...], preferred_element_type=jnp.float32)
        r_ref[...] = (jnp.dot(h, wr_ref[...], preferred_element_type=jnp.float32)
                      + b_ref[...])

    D = w_out.shape[1]
    return pl.pallas_call(
        body,
        grid=(N // BM,),
        in_specs=[
            pl.BlockSpec((2, BM, 128), lambda i: (0, i, 0)),
            pl.BlockSpec((BM, 16), lambda i: (i, 0)),
            pl.BlockSpec((BM, 128), lambda i: (i, 0)),
            pl.BlockSpec((128, D), lambda i: (0, 0)),
            pl.BlockSpec((128, D), lambda i: (0, 0)),
            pl.BlockSpec((1, D), lambda i: (0, 0)),
        ],
        out_specs=[pl.BlockSpec((BM, D), lambda i: (i, 0)),
                   pl.BlockSpec((BM, D), lambda i: (i, 0))],
        out_shape=[jax.ShapeDtypeStruct((N, D), jnp.float32)] * 2,
    )(p, dinv16, r_prev, w_out, w_root, b.reshape(1, D))


def _tc_final(p, dinv16, r_prev, d_out):
    def body(p_ref, dv_ref, rp_ref, o_ref):
        h = (p_ref[0] + p_ref[1]) * dv_ref[:, 0:1] + rp_ref[...]
        col = lax.broadcasted_iota(jnp.int32, h.shape, 1)
        hm = jnp.where(col < d_out, h, jnp.float32(-1e30))
        m = jnp.max(hm, axis=-1, keepdims=True)
        lse = jnp.log(jnp.sum(jnp.exp(hm - m), axis=-1, keepdims=True))
        o_ref[...] = (h - m - lse)[:, :d_out]

    D = p.shape[2]
    return pl.pallas_call(
        body,
        grid=(N // BM,),
        in_specs=[
            pl.BlockSpec((2, BM, D), lambda i: (0, i, 0)),
            pl.BlockSpec((BM, 16), lambda i: (i, 0)),
            pl.BlockSpec((BM, D), lambda i: (i, 0)),
        ],
        out_specs=pl.BlockSpec((BM, d_out), lambda i: (i, 0)),
        out_shape=jax.ShapeDtypeStruct((N, d_out), jnp.float32),
    )(p, dinv16, r_prev)


def kernel(x, edge_index, W_out_0, b_out_0, W_root_0, W_out_1, b_out_1,
           W_root_1, W_out_2, b_out_2, W_root_2):
    row2 = edge_index[0].reshape(E // CHUNK, CHUNK)
    col2 = edge_index[1].reshape(E // CHUNK, CHUNK)

    y0, r0 = _tc_first(x, W_out_0, W_root_0, b_out_0)
    p0, dega, degb = _sc_layer(128, True)(y0, row2, col2)
    y1, r1, dinv16 = _tc_mid1(p0, dega.reshape(NPAD, 1), degb.reshape(NPAD, 1),
                              r0, W_out_1, W_root_1, b_out_1)
    (p1,) = _sc_layer(128, False)(y1, row2, col2)
    w2o = jnp.pad(W_out_2, ((0, 0), (0, 64)))
    w2r = jnp.pad(W_root_2, ((0, 0), (0, 64)))
    b2 = jnp.pad(b_out_2, (0, 64))
    y2, r2 = _tc_mid2(p1, dinv16, r1, w2o, w2r, b2)
    (p2,) = _sc_layer(128, False)(y2, row2, col2)
    return _tc_final(p2, dinv16, r2, 64)


# speedup vs baseline: 1.1615x; 1.1449x over previous
import functools

import jax
import jax.numpy as jnp
from jax import lax
from jax.experimental import pallas as pl
from jax.experimental.pallas import tpu as pltpu
from jax.experimental.pallas import tpu_sc as plsc

N = 10000
NPAD = 10240
E = 320000
NCORES = 2
NSUB = 16
NTILES = NCORES * NSUB
EPT = E // NTILES
CHUNK = 128
NFULL = EPT // CHUNK
TAIL = EPT - NFULL * CHUNK
RPT = NPAD // NSUB
ZROWS = 128
ZCH = RPT // ZROWS

BM = 2000


def _fill2d(ref, nrows, ncols, value):
    per_row = ncols // 16
    vec = jnp.full((16,), value, dtype=jnp.float32)

    def body(i, c):
        r = i // per_row
        k = (i % per_row) * 16
        ref[r, pl.ds(k, 16)] = vec
        return c

    lax.fori_loop(0, nrows * per_row, body, 0)


def _fill1d(ref, n, value):
    vec = jnp.full((16,), value, dtype=jnp.float32)

    def body(i, c):
        ref[pl.ds(i * 16, 16)] = vec
        return c

    lax.fori_loop(0, n // 16, body, 0)


def _make_sc_layer(D, with_deg):
    def body(*refs):
        if with_deg:
            (y_hbm, row_hbm, col_hbm, part_hbm, dega_hbm, degb_hbm,
             agg_sh, deg_sh, rows0_v, rows1_v, ridx_v, cx0_v, cx1_v, tidx_v,
             ones_v, zdeg_v, gsem0, gsem1, hsem0, hsem1, csem0, csem1) = refs
        else:
            (y_hbm, row_hbm, col_hbm, part_hbm,
             agg_sh, rows0_v, rows1_v, ridx_v, cx0_v, cx1_v, tidx_v,
             gsem0, gsem1, hsem0, hsem1, csem0, csem1) = refs
        rows_v = rows0_v

        cid = lax.axis_index("c")
        sid = lax.axis_index("s")
        wid = cid * NSUB + sid
        ebase = wid * EPT

        _fill2d(rows_v, ZROWS, D, 0.0)
        for k in range(ZCH):
            pltpu.async_copy(
                rows_v, agg_sh.at[pl.ds(sid * RPT + k * ZROWS, ZROWS)], gsem0)
        if with_deg:
            _fill1d(ones_v, CHUNK, 1.0)
            _fill1d(zdeg_v, RPT, 0.0)
            pltpu.async_copy(zdeg_v, deg_sh.at[pl.ds(sid * RPT, RPT)], gsem1)
        pltpu.sync_copy(row_hbm.at[pl.ds(ebase, EPT)], ridx_v)
        for k in range(ZCH):
            pltpu.make_async_copy(
                rows_v, agg_sh.at[pl.ds(sid * RPT + k * ZROWS, ZROWS)],
                gsem0).wait()
        if with_deg:
            pltpu.make_async_copy(
                zdeg_v, deg_sh.at[pl.ds(sid * RPT, RPT)], gsem1).wait()
        plsc.subcore_barrier()

        rows = (rows0_v, rows1_v)
        rowsA = (rows0_v.at[pl.ds(0, 64)], rows1_v.at[pl.ds(0, 64)])
        rowsB = (rows0_v.at[pl.ds(64, 64)], rows1_v.at[pl.ds(64, 64)])
        gsems = (gsem0, gsem1)
        hsems = (hsem0, hsem1)
        cxs = (cx0_v, cx1_v)
        csems = (csem0, csem1)

        def gidx(j, o):
            return ridx_v.at[pl.ds(pl.multiple_of(j * CHUNK + o, 64), 64)]

        def issue_gather(j, b):
            pltpu.async_copy(y_hbm.at[gidx(j, 0)], rowsA[b], gsems[b])
            pltpu.async_copy(y_hbm.at[gidx(j, 64)], rowsB[b], hsems[b])
            pltpu.async_copy(
                col_hbm.at[pl.ds(pl.multiple_of(ebase + j * CHUNK, 8), CHUNK)],
                cxs[b], csems[b])

        def finish_chunk(j, b, src_rows):
            pltpu.make_async_copy(y_hbm.at[gidx(j, 0)], rowsA[b],
                                  gsems[b]).wait()
            pltpu.make_async_copy(y_hbm.at[gidx(j, 64)], rowsB[b],
                                  hsems[b]).wait()
            pltpu.make_async_copy(
                col_hbm.at[pl.ds(pl.multiple_of(ebase + j * CHUNK, 8), CHUNK)],
                cxs[b], csems[b]).wait()
            pltpu.sync_copy(src_rows, agg_sh.at[cxs[b]], add=True)
            if with_deg:
                pltpu.sync_copy(ones_v, deg_sh.at[cxs[b]], add=True)

        issue_gather(0, 0)

        def pair(i, c):
            j = 2 * i
            issue_gather(j + 1, 1)
            finish_chunk(j, 0, rows[0])
            issue_gather(j + 2, 0)
            finish_chunk(j + 1, 1, rows[1])
            return c

        lax.fori_loop(0, NFULL // 2 - 1, pair, 0)
        issue_gather(NFULL - 1, 1)
        finish_chunk(NFULL - 2, 0, rows[0])
        finish_chunk(NFULL - 1, 1, rows[1])

        tbase = ebase + NFULL * CHUNK
        pltpu.sync_copy(col_hbm.at[pl.ds(pl.multiple_of(tbase, 8), TAIL)],
                        tidx_v)
        pltpu.async_copy(
            y_hbm.at[ridx_v.at[pl.ds(pl.multiple_of(NFULL * CHUNK, 8), TAIL)]],
            rows0_v.at[pl.ds(0, TAIL)], gsem0)
        pltpu.make_async_copy(
            y_hbm.at[ridx_v.at[pl.ds(pl.multiple_of(NFULL * CHUNK, 8), TAIL)]],
            rows0_v.at[pl.ds(0, TAIL)], gsem0).wait()
        pltpu.sync_copy(rows0_v.at[pl.ds(0, TAIL)], agg_sh.at[tidx_v], add=True)
        if with_deg:
            pltpu.sync_copy(ones_v.at[pl.ds(0, TAIL)], deg_sh.at[tidx_v],
                            add=True)
        plsc.subcore_barrier()

        pltpu.sync_copy(agg_sh.at[pl.ds(sid * RPT, RPT)],
                        part_hbm.at[cid, pl.ds(sid * RPT, RPT)])
        if with_deg:
            @pl.when(cid == 0)
            def _():
                pltpu.sync_copy(deg_sh.at[pl.ds(sid * RPT, RPT)],
                                dega_hbm.at[pl.ds(sid * RPT, RPT)])

            @pl.when(cid == 1)
            def _():
                pltpu.sync_copy(deg_sh.at[pl.ds(sid * RPT, RPT)],
                                degb_hbm.at[pl.ds(sid * RPT, RPT)])

    out_type = [jax.ShapeDtypeStruct((NCORES, NPAD, D), jnp.float32)]
    scratch = [
        pltpu.VMEM_SHARED((NPAD, D), jnp.float32),
    ]
    if with_deg:
        out_type.append(jax.ShapeDtypeStruct((NPAD,), jnp.float32))
        out_type.append(jax.ShapeDtypeStruct((NPAD,), jnp.float32))
        scratch.append(pltpu.VMEM_SHARED((NPAD,), jnp.float32))
    scratch += [
        pltpu.VMEM((ZROWS, D), jnp.float32),
        pltpu.VMEM((ZROWS, D), jnp.float32),
        pltpu.VMEM((EPT,), jnp.int32),
        pltpu.VMEM((CHUNK,), jnp.int32),
        pltpu.VMEM((CHUNK,), jnp.int32),
        pltpu.VMEM((TAIL,), jnp.int32),
    ]
    if with_deg:
        scratch += [
            pltpu.VMEM((CHUNK,), jnp.float32),
            pltpu.VMEM((RPT,), jnp.float32),
        ]
    for _ in range(6):
        scratch.append(pltpu.SemaphoreType.DMA)

    mesh = plsc.VectorSubcoreMesh(core_axis_name="c", subcore_axis_name="s",
                                  num_cores=NCORES, num_subcores=NSUB)
    return pl.kernel(body, out_type=out_type, mesh=mesh, scratch_types=scratch)


@functools.lru_cache(maxsize=None)
def _sc_layer(D, with_deg):
    return _make_sc_layer(D, with_deg)


def _tc_first(x, w_out, w_root, b):
    def body(x_ref, wo_ref, wr_ref, b_ref, y_ref, r_ref):
        xb = x_ref[...]
        y_ref[...] = jnp.dot(xb, wo_ref[...], preferred_element_type=jnp.float32)
        r_ref[...] = (jnp.dot(xb, wr_ref[...], preferred_element_type=jnp.float32)
                      + b_ref[...])

    D = w_out.shape[1]
    return pl.pallas_call(
        body,
        grid=(N // BM,),
        in_specs=[
            pl.BlockSpec((BM, 128), lambda i: (i, 0)),
            pl.BlockSpec((128, D), lambda i: (0, 0)),
            pl.BlockSpec((128, D), lambda i: (0, 0)),
            pl.BlockSpec((1, D), lambda i: (0, 0)),
        ],
        out_specs=[pl.BlockSpec((BM, D), lambda i: (i, 0)),
                   pl.BlockSpec((BM, D), lambda i: (i, 0))],
        out_shape=[jax.ShapeDtypeStruct((N, D), jnp.float32)] * 2,
    )(x, w_out, w_root, b.reshape(1, D))


def _tc_mid1(p, dega, degb, r_prev, w_out, w_root, b):
    def body(p_ref, da_ref, db_ref, rp_ref, wo_ref, wr_ref, b_ref,
             y_ref, r_ref, dinv_ref):
        deg = da_ref[...] + db_ref[...]
        dinv = 1.0 / jnp.maximum(deg, 1.0)
        dinv_ref[...] = jnp.broadcast_to(dinv, dinv_ref.shape)
        h = jnp.maximum((p_ref[0] + p_ref[1]) * dinv + rp_ref[...], 0.0)
        y_ref[...] = jnp.dot(h, wo_ref[...], preferred_element_type=jnp.float32)
        r_ref[...] = (jnp.dot(h, wr_ref[...], preferred_element_type=jnp.float32)
                      + b_ref[...])

    D = w_out.shape[1]
    return pl.pallas_call(
        body,
        grid=(N // BM,),
        in_specs=[
            pl.BlockSpec((2, BM, 128), lambda i: (0, i, 0)),
            pl.BlockSpec((BM, 1), lambda i: (i, 0)),
            pl.BlockSpec((BM, 1), lambda i: (i, 0)),
            pl.BlockSpec((BM, 128), lambda i: (i, 0)),
            pl.BlockSpec((128, D), lambda i: (0, 0)),
            pl.BlockSpec((128, D), lambda i: (0, 0)),
            pl.BlockSpec((1, D), lambda i: (0, 0)),
        ],
        out_specs=[pl.BlockSpec((BM, D), lambda i: (i, 0)),
                   pl.BlockSpec((BM, D), lambda i: (i, 0)),
                   pl.BlockSpec((BM, 16), lambda i: (i, 0))],
        out_shape=[jax.ShapeDtypeStruct((N, D), jnp.float32),
                   jax.ShapeDtypeStruct((N, D), jnp.float32),
                   jax.ShapeDtypeStruct((N, 16), jnp.float32)],
    )(p, dega, degb, r_prev, w_out, w_root, b.reshape(1, D))


def _tc_mid2(p, dinv16, r_prev, w_out, w_root, b):
    def body(p_ref, dv_ref, rp_ref, wo_ref, wr_ref, b_ref, y_ref, r_ref):
        dinv = dv_ref[:, 0:1]
        h = jnp.maximum((p_ref[0] + p_ref[1]) * dinv + rp_ref[...], 0.0)
        y_ref[...] = jnp.dot(h, wo_ref[...], preferred_element_type=jnp.float32)
        r_ref[...] = (jnp.dot(h, wr_ref[...], preferred_element_type=jnp.float32)
                      + b_ref[...])

    D = w_out.shape[1]
    return pl.pallas_call(
        body,
        grid=(N // BM,),
        in_specs=[
            pl.BlockSpec((2, BM, 128), lambda i: (0, i, 0)),
            pl.BlockSpec((BM, 16), lambda i: (i, 0)),
            pl.BlockSpec((BM, 128), lambda i: (i, 0)),
            pl.BlockSpec((128, D), lambda i: (0, 0)),
            pl.BlockSpec((128, D), lambda i: (0, 0)),
            pl.BlockSpec((1, D), lambda i: (0, 0)),
        ],
        out_specs=[pl.BlockSpec((BM, D), lambda i: (i, 0)),
                   pl.BlockSpec((BM, D), lambda i: (i, 0))],
        out_shape=[jax.ShapeDtypeStruct((N, D), jnp.float32)] * 2,
    )(p, dinv16, r_prev, w_out, w_root, b.reshape(1, D))


def _tc_final(p, dinv16, r_prev, d_out):
    def body(p_ref, dv_ref, rp_ref, o_ref):
        h = (p_ref[0] + p_ref[1]) * dv_ref[:, 0:1] + rp_ref[...]
        col = lax.broadcasted_iota(jnp.int32, h.shape, 1)
        hm = jnp.where(col < d_out, h, jnp.float32(-1e30))
        m = jnp.max(hm, axis=-1, keepdims=True)
        lse = jnp.log(jnp.sum(jnp.exp(hm - m), axis=-1, keepdims=True))
        o_ref[...] = (h - m - lse)[:, :d_out]

    D = p.shape[2]
    return pl.pallas_call(
        body,
        grid=(N // BM,),
        in_specs=[
            pl.BlockSpec((2, BM, D), lambda i: (0, i, 0)),
            pl.BlockSpec((BM, 16), lambda i: (i, 0)),
            pl.BlockSpec((BM, D), lambda i: (i, 0)),
        ],
        out_specs=pl.BlockSpec((BM, d_out), lambda i: (i, 0)),
        out_shape=jax.ShapeDtypeStruct((N, d_out), jnp.float32),
    )(p, dinv16, r_prev)


def kernel(x, edge_index, W_out_0, b_out_0, W_root_0, W_out_1, b_out_1,
           W_root_1, W_out_2, b_out_2, W_root_2):
    row1 = edge_index[0]
    col1 = edge_index[1]

    y0, r0 = _tc_first(x, W_out_0, W_root_0, b_out_0)
    p0, dega, degb = _sc_layer(128, True)(y0, row1, col1)
    y1, r1, dinv16 = _tc_mid1(p0, dega.reshape(NPAD, 1), degb.reshape(NPAD, 1),
                              r0, W_out_1, W_root_1, b_out_1)
    (p1,) = _sc_layer(128, False)(y1, row1, col1)
    w2o = jnp.pad(W_out_2, ((0, 0), (0, 64)))
    w2r = jnp.pad(W_root_2, ((0, 0), (0, 64)))
    b2 = jnp.pad(b_out_2, (0, 64))
    y2, r2 = _tc_mid2(p1, dinv16, r1, w2o, w2r, b2)
    (p2,) = _sc_layer(128, False)(y2, row1, col1)
    return _tc_final(p2, dinv16, r2, 64)


# speedup vs baseline: 1.1633x; 1.0015x over previous
import functools

import jax
import jax.numpy as jnp
from jax import lax
from jax.experimental import pallas as pl
from jax.experimental.pallas import tpu as pltpu
from jax.experimental.pallas import tpu_sc as plsc

N = 10000
NPAD = 10240
E = 320000
NCORES = 2
NSUB = 16
NTILES = NCORES * NSUB
EPT = E // NTILES
CHUNK = 128
NFULL = EPT // CHUNK
TAIL = EPT - NFULL * CHUNK
RPT = NPAD // NSUB
ZROWS = 128
ZCH = RPT // ZROWS

BM = 2000


def _fill2d(ref, nrows, ncols, value):
    per_row = ncols // 16
    vec = jnp.full((16,), value, dtype=jnp.float32)

    def body(i, c):
        r = i // per_row
        k = (i % per_row) * 16
        ref[r, pl.ds(k, 16)] = vec
        return c

    lax.fori_loop(0, nrows * per_row, body, 0)


def _fill1d(ref, n, value):
    vec = jnp.full((16,), value, dtype=jnp.float32)

    def body(i, c):
        ref[pl.ds(i * 16, 16)] = vec
        return c

    lax.fori_loop(0, n // 16, body, 0)


def _make_sc_layer(D, with_deg):
    def body(*refs):
        if with_deg:
            (y_hbm, row_hbm, col_hbm, part_hbm, dega_hbm, degb_hbm,
             agg_sh, deg_sh, rows0_v, rows1_v, ridx_v, cx0_v, cx1_v, tidx_v,
             ones_v, zdeg_v, gsem0, gsem1, hsem0, hsem1, csem0, csem1) = refs
        else:
            (y_hbm, row_hbm, col_hbm, part_hbm,
             agg_sh, rows0_v, rows1_v, ridx_v, cx0_v, cx1_v, tidx_v,
             gsem0, gsem1, hsem0, hsem1, csem0, csem1) = refs
        rows_v = rows0_v

        cid = lax.axis_index("c")
        sid = lax.axis_index("s")
        wid = cid * NSUB + sid
        ebase = wid * EPT

        _fill2d(rows_v, ZROWS, D, 0.0)
        for k in range(ZCH):
            pltpu.async_copy(
                rows_v, agg_sh.at[pl.ds(sid * RPT + k * ZROWS, ZROWS)], gsem0)
        if with_deg:
            _fill1d(ones_v, CHUNK, 1.0)
            _fill1d(zdeg_v, RPT, 0.0)
            pltpu.async_copy(zdeg_v, deg_sh.at[pl.ds(sid * RPT, RPT)], gsem1)
        pltpu.sync_copy(row_hbm.at[pl.ds(ebase, EPT)], ridx_v)
        for k in range(ZCH):
            pltpu.make_async_copy(
                rows_v, agg_sh.at[pl.ds(sid * RPT + k * ZROWS, ZROWS)],
                gsem0).wait()
        if with_deg:
            pltpu.make_async_copy(
                zdeg_v, deg_sh.at[pl.ds(sid * RPT, RPT)], gsem1).wait()

        rows = (rows0_v, rows1_v)
        rowsA = (rows0_v.at[pl.ds(0, 64)], rows1_v.at[pl.ds(0, 64)])
        rowsB = (rows0_v.at[pl.ds(64, 64)], rows1_v.at[pl.ds(64, 64)])
        gsems = (gsem0, gsem1)
        hsems = (hsem0, hsem1)
        cxs = (cx0_v, cx1_v)
        csems = (csem0, csem1)

        def gidx(j, o):
            return ridx_v.at[pl.ds(pl.multiple_of(j * CHUNK + o, 64), 64)]

        def issue_gather(j, b):
            pltpu.async_copy(y_hbm.at[gidx(j, 0)], rowsA[b], gsems[b])
            pltpu.async_copy(y_hbm.at[gidx(j, 64)], rowsB[b], hsems[b])
            pltpu.async_copy(
                col_hbm.at[pl.ds(pl.multiple_of(ebase + j * CHUNK, 8), CHUNK)],
                cxs[b], csems[b])

        def finish_chunk(j, b, src_rows):
            pltpu.make_async_copy(y_hbm.at[gidx(j, 0)], rowsA[b],
                                  gsems[b]).wait()
            pltpu.make_async_copy(y_hbm.at[gidx(j, 64)], rowsB[b],
                                  hsems[b]).wait()
            pltpu.make_async_copy(
                col_hbm.at[pl.ds(pl.multiple_of(ebase + j * CHUNK, 8), CHUNK)],
                cxs[b], csems[b]).wait()
            pltpu.sync_copy(src_rows, agg_sh.at[cxs[b]], add=True)
            if with_deg:
                pltpu.sync_copy(ones_v, deg_sh.at[cxs[b]], add=True)

        issue_gather(0, 0)
        issue_gather(1, 1)
        plsc.subcore_barrier()

        def pair(i, c):
            j = 2 * i
            finish_chunk(j, 0, rows[0])
            issue_gather(j + 2, 0)
            finish_chunk(j + 1, 1, rows[1])
            issue_gather(j + 3, 1)
            return c

        lax.fori_loop(0, NFULL // 2 - 2, pair, 0)
        finish_chunk(NFULL - 4, 0, rows[0])
        issue_gather(NFULL - 2, 0)
        finish_chunk(NFULL - 3, 1, rows[1])
        issue_gather(NFULL - 1, 1)
        finish_chunk(NFULL - 2, 0, rows[0])
        finish_chunk(NFULL - 1, 1, rows[1])

        tbase = ebase + NFULL * CHUNK
        pltpu.sync_copy(col_hbm.at[pl.ds(pl.multiple_of(tbase, 8), TAIL)],
                        tidx_v)
        pltpu.async_copy(
            y_hbm.at[ridx_v.at[pl.ds(pl.multiple_of(NFULL * CHUNK, 8), TAIL)]],
            rows0_v.at[pl.ds(0, TAIL)], gsem0)
        pltpu.make_async_copy(
            y_hbm.at[ridx_v.at[pl.ds(pl.multiple_of(NFULL * CHUNK, 8), TAIL)]],
            rows0_v.at[pl.ds(0, TAIL)], gsem0).wait()
        pltpu.sync_copy(rows0_v.at[pl.ds(0, TAIL)], agg_sh.at[tidx_v], add=True)
        if with_deg:
            pltpu.sync_copy(ones_v.at[pl.ds(0, TAIL)], deg_sh.at[tidx_v],
                            add=True)
        plsc.subcore_barrier()

        pltpu.sync_copy(agg_sh.at[pl.ds(sid * RPT, RPT)],
                        part_hbm.at[cid, pl.ds(sid * RPT, RPT)])
        if with_deg:
            @pl.when(cid == 0)
            def _():
                pltpu.sync_copy(deg_sh.at[pl.ds(sid * RPT, RPT)],
                                dega_hbm.at[pl.ds(sid * RPT, RPT)])

            @pl.when(cid == 1)
            def _():
                pltpu.sync_copy(deg_sh.at[pl.ds(sid * RPT, RPT)],
                                degb_hbm.at[pl.ds(sid * RPT, RPT)])

    out_type = [jax.ShapeDtypeStruct((NCORES, NPAD, D), jnp.float32)]
    scratch = [
        pltpu.VMEM_SHARED((NPAD, D), jnp.float32),
    ]
    if with_deg:
        out_type.append(jax.ShapeDtypeStruct((NPAD,), jnp.float32))
        out_type.append(jax.ShapeDtypeStruct((NPAD,), jnp.float32))
        scratch.append(pltpu.VMEM_SHARED((NPAD,), jnp.float32))
    scratch += [
        pltpu.VMEM((ZROWS, D), jnp.float32),
        pltpu.VMEM((ZROWS, D), jnp.float32),
        pltpu.VMEM((EPT,), jnp.int32),
        pltpu.VMEM((CHUNK,), jnp.int32),
        pltpu.VMEM((CHUNK,), jnp.int32),
        pltpu.VMEM((TAIL,), jnp.int32),
    ]
    if with_deg:
        scratch += [
            pltpu.VMEM((CHUNK,), jnp.float32),
            pltpu.VMEM((RPT,), jnp.float32),
        ]
    for _ in range(6):
        scratch.append(pltpu.SemaphoreType.DMA)

    mesh = plsc.VectorSubcoreMesh(core_axis_name="c", subcore_axis_name="s",
                                  num_cores=NCORES, num_subcores=NSUB)
    return pl.kernel(body, out_type=out_type, mesh=mesh, scratch_types=scratch)


@functools.lru_cache(maxsize=None)
def _sc_layer(D, with_deg):
    return _make_sc_layer(D, with_deg)


def _tc_first(x, w_out, w_root, b):
    def body(x_ref, wo_ref, wr_ref, b_ref, y_ref, r_ref):
        xb = x_ref[...]
        y_ref[...] = jnp.dot(xb, wo_ref[...], preferred_element_type=jnp.float32)
        r_ref[...] = (jnp.dot(xb, wr_ref[...], preferred_element_type=jnp.float32)
                      + b_ref[...])

    D = w_out.shape[1]
    return pl.pallas_call(
        body,
        grid=(N // BM,),
        in_specs=[
            pl.BlockSpec((BM, 128), lambda i: (i, 0)),
            pl.BlockSpec((128, D), lambda i: (0, 0)),
            pl.BlockSpec((128, D), lambda i: (0, 0)),
            pl.BlockSpec((1, D), lambda i: (0, 0)),
        ],
        out_specs=[pl.BlockSpec((BM, D), lambda i: (i, 0)),
                   pl.BlockSpec((BM, D), lambda i: (i, 0))],
        out_shape=[jax.ShapeDtypeStruct((N, D), jnp.float32)] * 2,
    )(x, w_out, w_root, b.reshape(1, D))


def _tc_mid1(p, dega, degb, r_prev, w_out, w_root, b):
    def body(p_ref, da_ref, db_ref, rp_ref, wo_ref, wr_ref, b_ref,
             y_ref, r_ref, dinv_ref):
        deg = da_ref[...] + db_ref[...]
        dinv = 1.0 / jnp.maximum(deg, 1.0)
        dinv_ref[...] = jnp.broadcast_to(dinv, dinv_ref.shape)
        h = jnp.maximum((p_ref[0] + p_ref[1]) * dinv + rp_ref[...], 0.0)
        y_ref[...] = jnp.dot(h, wo_ref[...], preferred_element_type=jnp.float32)
        r_ref[...] = (jnp.dot(h, wr_ref[...], preferred_element_type=jnp.float32)
                      + b_ref[...])

    D = w_out.shape[1]
    return pl.pallas_call(
        body,
        grid=(N // BM,),
        in_specs=[
            pl.BlockSpec((2, BM, 128), lambda i: (0, i, 0)),
            pl.BlockSpec((BM, 1), lambda i: (i, 0)),
            pl.BlockSpec((BM, 1), lambda i: (i, 0)),
            pl.BlockSpec((BM, 128), lambda i: (i, 0)),
            pl.BlockSpec((128, D), lambda i: (0, 0)),
            pl.BlockSpec((128, D), lambda i: (0, 0)),
            pl.BlockSpec((1, D), lambda i: (0, 0)),
        ],
        out_specs=[pl.BlockSpec((BM, D), lambda i: (i, 0)),
                   pl.BlockSpec((BM, D), lambda i: (i, 0)),
                   pl.BlockSpec((BM, 16), lambda i: (i, 0))],
        out_shape=[jax.ShapeDtypeStruct((N, D), jnp.float32),
                   jax.ShapeDtypeStruct((N, D), jnp.float32),
                   jax.ShapeDtypeStruct((N, 16), jnp.float32)],
    )(p, dega, degb, r_prev, w_out, w_root, b.reshape(1, D))


def _tc_mid2(p, dinv16, r_prev, w_out, w_root, b):
    def body(p_ref, dv_ref, rp_ref, wo_ref, wr_ref, b_ref, y_ref, r_ref):
        dinv = dv_ref[:, 0:1]
        h = jnp.maximum((p_ref[0] + p_ref[1]) * dinv + rp_ref[...], 0.0)
        y_ref[...] = jnp.dot(h, wo_ref[...], preferred_element_type=jnp.float32)
        r_ref[...] = (jnp.dot(h, wr_ref[...], preferred_element_type=jnp.float32)
                      + b_ref[...])

    D = w_out.shape[1]
    return pl.pallas_call(
        body,
        grid=(N // BM,),
        in_specs=[
            pl.BlockSpec((2, BM, 128), lambda i: (0, i, 0)),
            pl.BlockSpec((BM, 16), lambda i: (i, 0)),
            pl.BlockSpec((BM, 128), lambda i: (i, 0)),
            pl.BlockSpec((128, D), lambda i: (0, 0)),
            pl.BlockSpec((128, D), lambda i: (0, 0)),
            pl.BlockSpec((1, D), lambda i: (0, 0)),
        ],
        out_specs=[pl.BlockSpec((BM, D), lambda i: (i, 0)),
                   pl.BlockSpec((BM, D), lambda i: (i, 0))],
        out_shape=[jax.ShapeDtypeStruct((N, D), jnp.float32)] * 2,
    )(p, dinv16, r_prev, w_out, w_root, b.reshape(1, D))


def _tc_final(p, dinv16, r_prev, d_out):
    def body(p_ref, dv_ref, rp_ref, o_ref):
        h = (p_ref[0] + p_ref[1]) * dv_ref[:, 0:1] + rp_ref[...]
        col = lax.broadcasted_iota(jnp.int32, h.shape, 1)
        hm = jnp.where(col < d_out, h, jnp.float32(-1e30))
        m = jnp.max(hm, axis=-1, keepdims=True)
        lse = jnp.log(jnp.sum(jnp.exp(hm - m), axis=-1, keepdims=True))
        o_ref[...] = (h - m - lse)[:, :d_out]

    D = p.shape[2]
    return pl.pallas_call(
        body,
        grid=(N // BM,),
        in_specs=[
            pl.BlockSpec((2, BM, D), lambda i: (0, i, 0)),
            pl.BlockSpec((BM, 16), lambda i: (i, 0)),
            pl.BlockSpec((BM, D), lambda i: (i, 0)),
        ],
        out_specs=pl.BlockSpec((BM, d_out), lambda i: (i, 0)),
        out_shape=jax.ShapeDtypeStruct((N, d_out), jnp.float32),
    )(p, dinv16, r_prev)


def kernel(x, edge_index, W_out_0, b_out_0, W_root_0, W_out_1, b_out_1,
           W_root_1, W_out_2, b_out_2, W_root_2):
    row1 = edge_index[0]
    col1 = edge_index[1]

    y0, r0 = _tc_first(x, W_out_0, W_root_0, b_out_0)
    p0, dega, degb = _sc_layer(128, True)(y0, row1, col1)
    y1, r1, dinv16 = _tc_mid1(p0, dega.reshape(NPAD, 1), degb.reshape(NPAD, 1),
                              r0, W_out_1, W_root_1, b_out_1)
    (p1,) = _sc_layer(128, False)(y1, row1, col1)
    w2o = jnp.pad(W_out_2, ((0, 0), (0, 64)))
    w2r = jnp.pad(W_root_2, ((0, 0), (0, 64)))
    b2 = jnp.pad(b_out_2, (0, 64))
    y2, r2 = _tc_mid2(p1, dinv16, r1, w2o, w2r, b2)
    (p2,) = _sc_layer(128, False)(y2, row1, col1)
    return _tc_final(p2, dinv16, r2, 64)
